# Initial kernel scaffold; baseline (speedup 1.0000x reference)
import jax
import jax.numpy as jnp
from jax.experimental import pallas as pl


def kernel(x, edge_index, W1_rel, b1, W1_root, W2_rel, b2, W2_root, Wl, bl):
    raise NotImplementedError("write your pallas kernel here")


# speedup vs baseline: 5.2828x; 5.2828x over previous
import jax
import jax.numpy as jnp
from jax import lax
from jax.experimental import pallas as pl
from jax.experimental.pallas import tpu as pltpu
from jax.experimental.pallas import tpu_sc as plsc

N_NODES = 10000
N_EDGES = 320000
D = 128
NC = 2
NS = 16
NW = NC * NS
EP = N_EDGES // NW
CHUNK = 80
NCH = EP // CHUNK
WT = 10
WROWS = N_NODES // WT
RQ = 200


def _agg_body(y_hbm, src_hbm, dst_hbm, out_hbm,
              src_v, dst_v, rows_v, zb, acc_sh, sem):
    z16 = jnp.zeros((16,), jnp.float32)
    c = lax.axis_index("c")
    s = lax.axis_index("s")
    wid = s * NC + c

    def zrow(i, carry):
        for j in range(D // 16):
            zb[i, pl.ds(j * 16, 16)] = z16
        return carry
    lax.fori_loop(0, RQ, zrow, 0)

    @pl.when(s < WT)
    def _():
        for q in range(WROWS // RQ):
            pltpu.sync_copy(zb, acc_sh.at[pl.ds(s * WROWS + q * RQ, RQ)])
    plsc.subcore_barrier()

    ebase = wid * EP

    def ebody(i, carry):
        base = ebase + i * CHUNK
        pltpu.sync_copy(src_hbm.at[pl.ds(base, CHUNK)], src_v)
        pltpu.sync_copy(dst_hbm.at[pl.ds(base, CHUNK)], dst_v)
        pltpu.async_copy(y_hbm.at[src_v], rows_v, sem).wait()
        pltpu.sync_copy(rows_v, acc_sh.at[dst_v], add=True)
        return carry
    lax.fori_loop(0, NCH, ebody, 0)
    plsc.subcore_barrier()

    @pl.when(s < WT)
    def _():
        for q in range(WROWS // RQ):
            r = s * WROWS + q * RQ
            pltpu.sync_copy(acc_sh.at[pl.ds(r, RQ)], zb)
            pltpu.sync_copy(zb, out_hbm.at[c, pl.ds(r, RQ)])


def _make_agg():
    return pl.kernel(
        _agg_body,
        out_type=jax.ShapeDtypeStruct((NC, N_NODES, D), jnp.float32),
        mesh=plsc.VectorSubcoreMesh(core_axis_name="c", subcore_axis_name="s"),
        scratch_types=[
            pltpu.VMEM((CHUNK,), jnp.int32),
            pltpu.VMEM((CHUNK,), jnp.int32),
            pltpu.VMEM((CHUNK, D), jnp.float32),
            pltpu.VMEM((RQ, D), jnp.float32),
            pltpu.VMEM_SHARED((N_NODES, D), jnp.float32),
            pltpu.SemaphoreType.DMA,
        ],
    )


_BM = 2000


def _mm1_body(x_ref, wrel_ref, wroot_ref, t1_ref, r1_ref):
    x = x_ref[...]
    y = jnp.dot(x, wrel_ref[...], preferred_element_type=jnp.float32)
    t1_ref[...] = jnp.concatenate(
        [y, jnp.ones((_BM, 1), jnp.float32),
         jnp.zeros((_BM, 63), jnp.float32)], axis=1)
    r1_ref[...] = jnp.dot(x, wroot_ref[...], preferred_element_type=jnp.float32)


def _mm1(x, w1rel_t, w1root_t):
    return pl.pallas_call(
        _mm1_body,
        grid=(N_NODES // _BM,),
        in_specs=[pl.BlockSpec((_BM, 128), lambda i: (i, 0)),
                  pl.BlockSpec((128, 64), lambda i: (0, 0)),
                  pl.BlockSpec((128, 64), lambda i: (0, 0))],
        out_specs=[pl.BlockSpec((_BM, D), lambda i: (i, 0)),
                   pl.BlockSpec((_BM, 64), lambda i: (i, 0))],
        out_shape=[jax.ShapeDtypeStruct((N_NODES, D), jnp.float32),
                   jax.ShapeDtypeStruct((N_NODES, 64), jnp.float32)],
    )(x, w1rel_t, w1root_t)


def _combine1_body(p0_ref, p1_ref, r1_ref, b1_ref, wrel_ref, wroot_ref,
                   t2_ref):
    p0 = p0_ref[...]
    p1 = p1_ref[...]
    inv = 1.0 / jnp.maximum(p0[:, 64:65] + p1[:, 64:65], 1.0)
    h = jnp.maximum(
        (p0[:, :64] + p1[:, :64]) * inv + b1_ref[...] + r1_ref[...], 0.0)
    y2 = jnp.dot(h, wrel_ref[...], preferred_element_type=jnp.float32)
    r2 = jnp.dot(h, wroot_ref[...], preferred_element_type=jnp.float32)
    t2_ref[...] = jnp.concatenate(
        [y2, r2, inv, jnp.zeros((_BM, 63), jnp.float32)], axis=1)


def _combine1(p0, p1, r1, b1r, w2rel_t, w2root_t):
    return pl.pallas_call(
        _combine1_body,
        grid=(N_NODES // _BM,),
        in_specs=[pl.BlockSpec((_BM, D), lambda i: (i, 0)),
                  pl.BlockSpec((_BM, D), lambda i: (i, 0)),
                  pl.BlockSpec((_BM, 64), lambda i: (i, 0)),
                  pl.BlockSpec((1, 64), lambda i: (0, 0)),
                  pl.BlockSpec((64, 32), lambda i: (0, 0)),
                  pl.BlockSpec((64, 32), lambda i: (0, 0))],
        out_specs=pl.BlockSpec((_BM, D), lambda i: (i, 0)),
        out_shape=jax.ShapeDtypeStruct((N_NODES, D), jnp.float32),
    )(p0, p1, r1, b1r, w2rel_t, w2root_t)


def _combine2_body(p0_ref, p1_ref, t2_ref, b2_ref, wl_ref, bl_ref, o_ref):
    p0 = p0_ref[...]
    p1 = p1_ref[...]
    t2 = t2_ref[...]
    inv = t2[:, 64:65]
    h = jnp.maximum(
        (p0[:, :32] + p1[:, :32]) * inv + b2_ref[...] + t2[:, 32:64], 0.0)
    o_ref[...] = jnp.dot(h, wl_ref[...],
                         preferred_element_type=jnp.float32) + bl_ref[...]


def _combine2(p0, p1, t2, b2r, wl_t, blr):
    return pl.pallas_call(
        _combine2_body,
        grid=(N_NODES // _BM,),
        in_specs=[pl.BlockSpec((_BM, D), lambda i: (i, 0)),
                  pl.BlockSpec((_BM, D), lambda i: (i, 0)),
                  pl.BlockSpec((_BM, D), lambda i: (i, 0)),
                  pl.BlockSpec((1, 32), lambda i: (0, 0)),
                  pl.BlockSpec((32, 1), lambda i: (0, 0)),
                  pl.BlockSpec((1, 1), lambda i: (0, 0))],
        out_specs=pl.BlockSpec((_BM, 1), lambda i: (i, 0)),
        out_shape=jax.ShapeDtypeStruct((N_NODES, 1), jnp.float32),
    )(p0, p1, t2, b2r, wl_t, blr)


def kernel(x, edge_index, W1_rel, b1, W1_root, W2_rel, b2, W2_root, Wl, bl):
    src = edge_index[0].astype(jnp.int32)
    dst = edge_index[1].astype(jnp.int32)

    t1, r1 = _mm1(x, W1_rel.T, W1_root.T)
    agg = _make_agg()
    sums1 = agg(t1, src, dst)
    t2 = _combine1(sums1[0], sums1[1], r1, b1.reshape(1, 64),
                   W2_rel.T, W2_root.T)
    sums2 = agg(t2, src, dst)
    return _combine2(sums2[0], sums2[1], t2, b2.reshape(1, 32),
                     Wl.T, bl.reshape(1, 1))
